# Initial kernel scaffold; baseline (speedup 1.0000x reference)
import jax
import jax.numpy as jnp
from jax.experimental import pallas as pl


def kernel(x, embedding):
    raise NotImplementedError("write your pallas kernel here")


# speedup vs baseline: 4.2899x; 4.2899x over previous
import functools

import jax
import jax.numpy as jnp
from jax import lax
from jax.experimental import pallas as pl
from jax.experimental.pallas import tpu as pltpu
from jax.experimental.pallas import tpu_sc as plsc

_EMSIZE = 64
_CHUNK = 1024
_GATHER = 128
_LANES = 16


def _sc_embedding_gather(xf, embedding, num_embs, n_total):
    n_workers = 32
    per_worker = n_total // n_workers
    n_chunks = per_worker // _CHUNK
    mesh = plsc.VectorSubcoreMesh(core_axis_name="c", subcore_axis_name="s")

    @functools.partial(
        pl.kernel,
        mesh=mesh,
        out_type=jax.ShapeDtypeStruct((n_total, _EMSIZE), jnp.float32),
        scratch_types=[
            pltpu.VMEM((_CHUNK,), jnp.float32),
            pltpu.VMEM((_CHUNK // _GATHER, _GATHER), jnp.int32),
            pltpu.VMEM((_CHUNK, _EMSIZE), jnp.float32),
            pltpu.SemaphoreType.DMA,
        ],
        compiler_params=pltpu.CompilerParams(use_tc_tiling_on_sc=False),
    )
    def body(x_hbm, table_hbm, out_hbm, xbuf, idxbuf, rows, sem):
        wid = lax.axis_index("s") * 2 + lax.axis_index("c")
        base = wid * per_worker

        def chunk_body(c, carry):
            off = base + c * _CHUNK
            pltpu.sync_copy(x_hbm.at[pl.ds(off, _CHUNK)], xbuf)

            def row_body(j, carry2):
                def vec_body(k, carry3):
                    v = xbuf[pl.ds(j * _GATHER + k * _LANES, _LANES)]
                    bad = (v != v) | (jnp.abs(v) == jnp.inf)
                    cl = jnp.minimum(jnp.maximum(v, 0.0),
                                     float(num_embs - 2))
                    iv = jnp.where(bad, float(num_embs - 1), cl)
                    idxbuf[j, pl.ds(k * _LANES, _LANES)] = iv.astype(jnp.int32)
                    return carry3

                return lax.fori_loop(0, _GATHER // _LANES, vec_body, carry2)

            lax.fori_loop(0, _CHUNK // _GATHER, row_body, 0)

            copies = []
            for j in range(_CHUNK // _GATHER):
                copies.append(pltpu.async_copy(
                    table_hbm.at[idxbuf.at[j]],
                    rows.at[pl.ds(j * _GATHER, _GATHER)],
                    sem))
            for cp in copies:
                cp.wait()

            pltpu.sync_copy(rows, out_hbm.at[pl.ds(off, _CHUNK)])
            return carry

        lax.fori_loop(0, n_chunks, chunk_body, 0)

    return body(xf, embedding)


def kernel(x, embedding):
    t, b, _ = x.shape
    num_embs = embedding.shape[0]
    xf = x.reshape(t * b)
    out = _sc_embedding_gather(xf, embedding, num_embs, t * b)
    return out.reshape(t, b, _EMSIZE)


# speedup vs baseline: 4.4143x; 1.0290x over previous
import functools

import jax
import jax.numpy as jnp
from jax import lax
from jax.experimental import pallas as pl
from jax.experimental.pallas import tpu as pltpu
from jax.experimental.pallas import tpu_sc as plsc

_EMSIZE = 64
_CHUNK = 512
_GATHER = 128
_LANES = 16
_NBUF = 2


def _sc_embedding_gather(xf, embedding, num_embs, n_total):
    n_workers = 32
    per_worker = n_total // n_workers
    n_chunks = per_worker // _CHUNK
    n_g = _CHUNK // _GATHER
    mesh = plsc.VectorSubcoreMesh(core_axis_name="c", subcore_axis_name="s")

    @functools.partial(
        pl.kernel,
        mesh=mesh,
        out_type=jax.ShapeDtypeStruct((n_total, _EMSIZE), jnp.float32),
        scratch_types=[
            pltpu.VMEM((_NBUF, _CHUNK), jnp.float32),
            pltpu.VMEM((_NBUF, n_g, _GATHER), jnp.int32),
            pltpu.VMEM((_NBUF, _CHUNK, _EMSIZE), jnp.float32),
        ] + [pltpu.SemaphoreType.DMA] * (3 * _NBUF),
        compiler_params=pltpu.CompilerParams(use_tc_tiling_on_sc=False),
    )
    def body(x_hbm, table_hbm, out_hbm, xbuf, idxbuf, rows,
             xs0, xs1, gs0, gs1, os0, os1):
        xsem = (xs0, xs1)
        gsem = (gs0, gs1)
        osem = (os0, os1)
        wid = lax.axis_index("s") * 2 + lax.axis_index("c")
        base = wid * per_worker

        def transform(b, xoff):
            for j in range(n_g):
                def vec_body(k, carry):
                    v = xbuf[b, pl.ds(xoff + j * _GATHER + k * _LANES,
                                      _LANES)]
                    bad = (v != v) | (jnp.abs(v) == jnp.inf)
                    cl = jnp.minimum(jnp.maximum(v, 0.0),
                                     float(num_embs - 2))
                    iv = jnp.where(bad, float(num_embs - 1), cl)
                    idxbuf[b, j, pl.ds(k * _LANES, _LANES)] = (
                        iv.astype(jnp.int32))
                    return carry

                lax.fori_loop(0, _GATHER // _LANES, vec_body, 0,
                              unroll=4)

        def fire_gathers(b):
            for j in range(n_g):
                pltpu.async_copy(
                    table_hbm.at[idxbuf.at[b, j]],
                    rows.at[b, pl.ds(j * _GATHER, _GATHER)],
                    gsem[b])

        def group_body(g, carry):
            for b in range(_NBUF):
                c = g * _NBUF + b
                off = base + c * _CHUNK
                ob = 1 - b

                @pl.when(c > 0)
                def _():
                    pltpu.make_async_copy(
                        out_hbm.at[pl.ds(base, _CHUNK)],
                        rows.at[ob], gsem[ob]).wait()
                    pltpu.async_copy(
                        rows.at[ob],
                        out_hbm.at[pl.ds(off - _CHUNK, _CHUNK)],
                        osem[ob])

                @pl.when(c + 1 < n_chunks)
                def _():
                    pltpu.async_copy(
                        x_hbm.at[pl.ds(off + _CHUNK, _CHUNK)],
                        xbuf.at[ob], xsem[ob])

                pltpu.make_async_copy(
                    x_hbm.at[pl.ds(off, _CHUNK)],
                    xbuf.at[b], xsem[b]).wait()
                transform(b, 0)

                @pl.when(c >= _NBUF)
                def _():
                    pltpu.make_async_copy(
                        rows.at[b], out_hbm.at[pl.ds(base, _CHUNK)],
                        osem[b]).wait()

                fire_gathers(b)
            return carry

        pltpu.async_copy(x_hbm.at[pl.ds(base, _CHUNK)], xbuf.at[0], xsem[0])
        lax.fori_loop(0, n_chunks // _NBUF, group_body, 0)

        last = n_chunks - 1
        lb = last % _NBUF
        pltpu.make_async_copy(out_hbm.at[pl.ds(base, _CHUNK)],
                              rows.at[lb], gsem[lb]).wait()
        pltpu.async_copy(rows.at[lb],
                         out_hbm.at[pl.ds(base + last * _CHUNK, _CHUNK)],
                         osem[lb])
        pltpu.make_async_copy(rows.at[1 - lb],
                              out_hbm.at[pl.ds(base, _CHUNK)],
                              osem[1 - lb]).wait()
        pltpu.make_async_copy(rows.at[lb],
                              out_hbm.at[pl.ds(base, _CHUNK)],
                              osem[lb]).wait()

    return body(xf, embedding)


def kernel(x, embedding):
    t, b, _ = x.shape
    num_embs = embedding.shape[0]
    xf = x.reshape(t * b)
    out = _sc_embedding_gather(xf, embedding, num_embs, t * b)
    return out.reshape(t, b, _EMSIZE)


# speedup vs baseline: 4.4293x; 1.0034x over previous
import functools

import jax
import jax.numpy as jnp
from jax import lax
from jax.experimental import pallas as pl
from jax.experimental.pallas import tpu as pltpu
from jax.experimental.pallas import tpu_sc as plsc

_EMSIZE = 64
_CHUNK = 512
_GATHER = 128
_LANES = 16
_NBUF = 2


def _sc_embedding_gather(xf, embedding, num_embs, t_dim, b_dim):
    n_total = t_dim * b_dim
    n_workers = 32
    per_worker = n_total // n_workers
    n_chunks = per_worker // _CHUNK
    n_g = _CHUNK // _GATHER
    mesh = plsc.VectorSubcoreMesh(core_axis_name="c", subcore_axis_name="s")

    @functools.partial(
        pl.kernel,
        mesh=mesh,
        out_type=jax.ShapeDtypeStruct((t_dim, b_dim, _EMSIZE), jnp.float32),
        scratch_types=[
            pltpu.VMEM((_NBUF, _CHUNK), jnp.float32),
            pltpu.VMEM((_NBUF, n_g, _GATHER), jnp.int32),
            pltpu.VMEM((_NBUF, _CHUNK, _EMSIZE), jnp.float32),
        ] + [pltpu.SemaphoreType.DMA] * (3 * _NBUF),
        compiler_params=pltpu.CompilerParams(use_tc_tiling_on_sc=False),
    )
    def body(x_hbm, table_hbm, out_hbm, xbuf, idxbuf, rows,
             xs0, xs1, gs0, gs1, os0, os1):
        xsem = (xs0, xs1)
        gsem = (gs0, gs1)
        osem = (os0, os1)
        wid = lax.axis_index("s") * 2 + lax.axis_index("c")
        base = wid * per_worker

        def out_slice(off):
            return out_hbm.at[off // b_dim, pl.ds(off % b_dim, _CHUNK)]

        def transform(b):
            for j in range(n_g):
                def vec_body(k, carry):
                    v = xbuf[b, pl.ds(j * _GATHER + k * _LANES, _LANES)]
                    bad = (v != v) | (jnp.abs(v) == jnp.inf)
                    cl = jnp.minimum(jnp.maximum(v, 0.0),
                                     float(num_embs - 2))
                    iv = jnp.where(bad, float(num_embs - 1), cl)
                    idxbuf[b, j, pl.ds(k * _LANES, _LANES)] = (
                        iv.astype(jnp.int32))
                    return carry

                lax.fori_loop(0, _GATHER // _LANES, vec_body, 0,
                              unroll=4)

        def fire_gathers(b):
            for j in range(n_g):
                pltpu.async_copy(
                    table_hbm.at[idxbuf.at[b, j]],
                    rows.at[b, pl.ds(j * _GATHER, _GATHER)],
                    gsem[b])

        def group_body(g, carry):
            for b in range(_NBUF):
                c = g * _NBUF + b
                off = base + c * _CHUNK
                ob = 1 - b

                @pl.when(c > 0)
                def _():
                    pltpu.make_async_copy(
                        out_slice(base), rows.at[ob], gsem[ob]).wait()
                    pltpu.async_copy(
                        rows.at[ob], out_slice(off - _CHUNK), osem[ob])

                @pl.when(c + 1 < n_chunks)
                def _():
                    pltpu.async_copy(
                        x_hbm.at[pl.ds(off + _CHUNK, _CHUNK)],
                        xbuf.at[ob], xsem[ob])

                pltpu.make_async_copy(
                    x_hbm.at[pl.ds(off, _CHUNK)],
                    xbuf.at[b], xsem[b]).wait()
                transform(b)

                @pl.when(c >= _NBUF)
                def _():
                    pltpu.make_async_copy(
                        rows.at[b], out_slice(base), osem[b]).wait()

                fire_gathers(b)
            return carry

        pltpu.async_copy(x_hbm.at[pl.ds(base, _CHUNK)], xbuf.at[0], xsem[0])
        lax.fori_loop(0, n_chunks // _NBUF, group_body, 0)

        last = n_chunks - 1
        lb = last % _NBUF
        pltpu.make_async_copy(out_slice(base), rows.at[lb], gsem[lb]).wait()
        pltpu.async_copy(rows.at[lb], out_slice(base + last * _CHUNK),
                         osem[lb])
        pltpu.make_async_copy(rows.at[1 - lb], out_slice(base),
                              osem[1 - lb]).wait()
        pltpu.make_async_copy(rows.at[lb], out_slice(base),
                              osem[lb]).wait()

    return body(xf, embedding)


def kernel(x, embedding):
    t, b, _ = x.shape
    num_embs = embedding.shape[0]
    xf = x.reshape(t * b)
    return _sc_embedding_gather(xf, embedding, num_embs, t, b)
